# baseline (speedup 1.0000x reference)
import functools

import jax
import jax.numpy as jnp
from jax import lax
from jax.experimental import pallas as pl
from jax.experimental.pallas import tpu as pltpu
from jax.experimental.pallas import tpu_sc as plsc

_LANES = 16


def _make_kernel(B, T, V, D, NC, NS, C):
    NW = NC * NS
    TB = T // NW
    n_chunks = TB // C
    mesh = plsc.VectorSubcoreMesh(core_axis_name="c", subcore_axis_name="s")

    @functools.partial(
        pl.kernel,
        mesh=mesh,
        out_type=jax.ShapeDtypeStruct((B, T, D), jnp.float32),
        scratch_types=[
            pltpu.VMEM((C,), jnp.int32),
            pltpu.VMEM((C, D), jnp.float32),
            pltpu.VMEM((C, D), jnp.float32),
            pltpu.SemaphoreType.DMA,
        ],
    )
    def body(idx_hbm, tok_hbm, pos_hbm, out_hbm, idx_v, tok_v, pos_v, sem):
        wid = lax.axis_index("s") * NC + lax.axis_index("c")
        t0 = wid * TB
        for ch in range(n_chunks):
            tc0 = t0 + ch * C
            pltpu.sync_copy(pos_hbm.at[pl.ds(tc0, C)], pos_v)
            for b in range(B):
                pltpu.sync_copy(idx_hbm.at[b, pl.ds(tc0, C)], idx_v)
                pltpu.async_copy(tok_hbm.at[idx_v], tok_v, sem).wait()

                def row_body(r, _):
                    def col_body(j, _):
                        off = pl.multiple_of(j * _LANES, _LANES)
                        tok_v[r, pl.ds(off, _LANES)] = (
                            tok_v[r, pl.ds(off, _LANES)]
                            + pos_v[r, pl.ds(off, _LANES)]
                        )
                        return 0

                    return lax.fori_loop(0, D // _LANES, col_body, 0)

                lax.fori_loop(0, C, row_body, 0)
                pltpu.sync_copy(tok_v, out_hbm.at[b, pl.ds(tc0, C)])

    return body


def kernel(idx, token_emb, pos_emb):
    B, T = idx.shape
    V, D = token_emb.shape
    info = plsc.get_sparse_core_info()
    NC, NS = info.num_cores, info.num_subcores
    body = _make_kernel(B, T, V, D, NC, NS, C=32)
    return body(idx.astype(jnp.int32), token_emb, pos_emb)


# speedup vs baseline: 2.0237x; 2.0237x over previous
import functools

import jax
import jax.numpy as jnp
from jax import lax
from jax.experimental import pallas as pl
from jax.experimental.pallas import tpu as pltpu
from jax.experimental.pallas import tpu_sc as plsc

_LANES = 16


def _make_kernel(B, T, V, D, NC, NS, C):
    NW = NC * NS
    TB = T // NW
    n_chunks = TB // C
    nsteps = n_chunks * B
    mesh = plsc.VectorSubcoreMesh(core_axis_name="c", subcore_axis_name="s")

    @functools.partial(
        pl.kernel,
        mesh=mesh,
        out_type=jax.ShapeDtypeStruct((B, T, D), jnp.float32),
        scratch_types=[
            pltpu.VMEM((B, TB), jnp.int32),
            pltpu.VMEM((2, C, D), jnp.float32),
            pltpu.VMEM((C, D), jnp.float32),
            pltpu.SemaphoreType.DMA,
            pltpu.SemaphoreType.DMA,
            pltpu.SemaphoreType.DMA,
        ],
    )
    def body(idx_hbm, tok_hbm, pos_hbm, out_hbm, idx_all, tok_v, pos_v,
             sem_g, sem_s0, sem_s1):
        wid = lax.axis_index("s") * NC + lax.axis_index("c")
        t0 = wid * TB
        sem_s = (sem_s0, sem_s1)
        for b in range(B):
            pltpu.sync_copy(idx_hbm.at[b, pl.ds(t0, TB)], idx_all.at[b])

        def fire_gather(s):
            ch, b = divmod(s, B)
            return pltpu.async_copy(
                tok_hbm.at[idx_all.at[b, pl.ds(ch * C, C)]],
                tok_v.at[s % 2], sem_g)

        gather = fire_gather(0)
        stores = [None, None]
        for s in range(nsteps):
            ch, b = divmod(s, B)
            buf = s % 2
            if b == 0:
                pltpu.sync_copy(pos_hbm.at[pl.ds(t0 + ch * C, C)], pos_v)
            if s + 1 < nsteps:
                if stores[(s + 1) % 2] is not None:
                    stores[(s + 1) % 2].wait()
                gather_next = fire_gather(s + 1)
            gather.wait()

            def row_body(r, _):
                for jb in range(D // _LANES):
                    off = jb * _LANES
                    tok_v[buf, r, pl.ds(off, _LANES)] = (
                        tok_v[buf, r, pl.ds(off, _LANES)]
                        + pos_v[r, pl.ds(off, _LANES)]
                    )
                return 0

            lax.fori_loop(0, C, row_body, 0)
            stores[buf] = pltpu.async_copy(
                tok_v.at[buf], out_hbm.at[b, pl.ds(t0 + ch * C, C)],
                sem_s[buf])
            if s + 1 < nsteps:
                gather = gather_next
        stores[0].wait()
        stores[1].wait()

    return body


def kernel(idx, token_emb, pos_emb):
    B, T = idx.shape
    V, D = token_emb.shape
    info = plsc.get_sparse_core_info()
    NC, NS = info.num_cores, info.num_subcores
    body = _make_kernel(B, T, V, D, NC, NS, C=32)
    return body(idx.astype(jnp.int32), token_emb, pos_emb)
